# Initial kernel scaffold; baseline (speedup 1.0000x reference)
import functools

import jax
import jax.numpy as jnp
from jax import lax
from jax.experimental import pallas as pl
from jax.experimental.pallas import tpu as pltpu
from jax.experimental.pallas import tpu_sc as plsc

N = 50000
E = 100000
C = 5
NF = 64
SF = 16
P = 3

CK = 128
E_PAD = 102400
BN = 2000
BE = 2048
HALF = N // 2
TRASH = HALF

F32 = jnp.float32
INTERPRET = False


def _k1_body(m_ref, w_ref, t_ref, a_ref):
    x = m_ref[0]
    for c in range(C):
        y = jnp.dot(x[:, c, :], w_ref[0, c], preferred_element_type=F32)
        t_ref[0, :, c * SF:(c + 1) * SF] = y[:, :SF]
        a_ref[0, :, c * SF:(c + 1) * SF] = y[:, SF:]


def _k1(ms, wk1):
    return pl.pallas_call(
        _k1_body,
        grid=(P, N // BN),
        in_specs=[
            pl.BlockSpec((1, BN, C, NF), lambda p, i: (p, i, 0, 0)),
            pl.BlockSpec((1, C, NF, 2 * SF), lambda p, i: (p, 0, 0, 0)),
        ],
        out_specs=[
            pl.BlockSpec((1, BN, C * SF), lambda p, i: (p, i, 0)),
            pl.BlockSpec((1, BN, C * SF), lambda p, i: (p, i, 0)),
        ],
        out_shape=[
            jax.ShapeDtypeStruct((P, N, C * SF), F32),
            jax.ShapeDtypeStruct((P, N, C * SF), F32),
        ],
        interpret=INTERPRET,
    )(ms, wk1)


def _k3_body(z_ref, b3a_ref, w3b_ref, b3b_ref, wbot_ref, be1_ref, tab_ref):
    h = jnp.tanh(z_ref[...] + b3a_ref[...])
    msp = []
    for c in range(C):
        hc = h[:, c * SF:(c + 1) * SF]
        msp.append(jnp.tanh(
            jnp.dot(hc, w3b_ref[c], preferred_element_type=F32)
            + b3b_ref[0, c * SF:(c + 1) * SF]))
    msp = jnp.concatenate(msp, axis=1)
    for p in range(P):
        bp = []
        for c in range(C):
            bp.append(
                jnp.dot(msp[:, c * SF:(c + 1) * SF], wbot_ref[p, c],
                        preferred_element_type=F32)
                + be1_ref[p, c * SF:(c + 1) * SF])
        tab_ref[p, :, :C * SF] = jnp.concatenate(bp, axis=1)
        tab_ref[p, :, C * SF:] = msp


def _k3(z, b3a_flat, w3b, b3b_flat, wbot, be1_flat):
    return pl.pallas_call(
        _k3_body,
        grid=(N // BN,),
        in_specs=[
            pl.BlockSpec((BN, C * SF), lambda i: (i, 0)),
            pl.BlockSpec((1, C * SF), lambda i: (0, 0)),
            pl.BlockSpec((C, SF, SF), lambda i: (0, 0, 0)),
            pl.BlockSpec((1, C * SF), lambda i: (0, 0)),
            pl.BlockSpec((P, C, SF, SF), lambda i: (0, 0, 0, 0)),
            pl.BlockSpec((P, C * SF), lambda i: (0, 0)),
        ],
        out_specs=pl.BlockSpec((P, BN, 2 * C * SF), lambda i: (0, i, 0)),
        out_shape=jax.ShapeDtypeStruct((P, N, 2 * C * SF), F32),
        interpret=INTERPRET,
    )(z, b3a_flat, w3b, b3b_flat, wbot, be1_flat)


def _k5_body(ga_ref, gs_ref, w2_ref, be2_ref, ebd_ref, msg_ref):
    ga = ga_ref[0]
    gs = gs_ref[0]
    e1 = jnp.tanh(ga + gs[:, :C * SF])
    logits = jnp.dot(e1, w2_ref[0], preferred_element_type=F32) + be2_ref[...]
    mx = jnp.max(logits, axis=1, keepdims=True)
    ex = jnp.exp(logits - mx)
    att = ex / jnp.sum(ex, axis=1, keepdims=True)
    expand = jnp.dot(att, ebd_ref[...], preferred_element_type=F32)
    msg_ref[0] = expand * gs[:, C * SF:]


def _k5(ga, gs, w2bd, be2, ebd):
    return pl.pallas_call(
        _k5_body,
        grid=(P, E_PAD // BE),
        in_specs=[
            pl.BlockSpec((1, BE, C * SF), lambda p, i: (p, i, 0)),
            pl.BlockSpec((1, BE, 2 * C * SF), lambda p, i: (p, i, 0)),
            pl.BlockSpec((1, C * SF, C), lambda p, i: (p, 0, 0)),
            pl.BlockSpec((1, C), lambda p, i: (p, 0)),
            pl.BlockSpec((C, C * SF), lambda p, i: (0, 0)),
        ],
        out_specs=pl.BlockSpec((1, BE, C * SF), lambda p, i: (p, i, 0)),
        out_shape=jax.ShapeDtypeStruct((P, E_PAD, C * SF), F32),
        interpret=INTERPRET,
    )(ga, gs, w2bd, be2, ebd)


def _k7_body(m_ref, ss_ref, cnt_ref, wn1_ref, bn1_ref, wn2_ref, bn2_ref, o_ref):
    m = m_ref[0]
    ss = ss_ref[0]
    cnt = jnp.clip(cnt_ref[0][:, 0:1], 1.0, None)
    outs = []
    for c in range(C):
        mean_c = ss[:, c * SF:(c + 1) * SF] / cnt
        mcat = jnp.concatenate([m[:, c, :], mean_c], axis=1)
        h = jnp.tanh(jnp.dot(mcat, wn1_ref[0, c], preferred_element_type=F32)
                     + bn1_ref[0, c])
        h = jnp.tanh(jnp.dot(h, wn2_ref[0, c], preferred_element_type=F32)
                     + bn2_ref[0, c])
        outs.append(h[:, None, :])
    o_ref[0] = jnp.concatenate(outs, axis=1)


def _k7(ms, ssum, cnt, wn1, bn1, wn2, bn2):
    return pl.pallas_call(
        _k7_body,
        grid=(P, N // BN),
        in_specs=[
            pl.BlockSpec((1, BN, C, NF), lambda p, i: (p, i, 0, 0)),
            pl.BlockSpec((1, BN, C * SF), lambda p, i: (p, i, 0)),
            pl.BlockSpec((1, BN, 16), lambda p, i: (p, i, 0)),
            pl.BlockSpec((1, C, NF + SF, NF), lambda p, i: (p, 0, 0, 0)),
            pl.BlockSpec((1, C, NF), lambda p, i: (p, 0, 0)),
            pl.BlockSpec((1, C, NF, NF), lambda p, i: (p, 0, 0, 0)),
            pl.BlockSpec((1, C, NF), lambda p, i: (p, 0, 0)),
        ],
        out_specs=pl.BlockSpec((1, BN, C, NF), lambda p, i: (p, i, 0, 0)),
        out_shape=jax.ShapeDtypeStruct((P, N, C, NF), F32),
        interpret=INTERPRET,
    )(ms, ssum, cnt, wn1, bn1, wn2, bn2)


def kernel(m_u, m_v, m_y, edge_index_u, edge_index_v, edge_index_y,
           W3a, b3a, W3b, b3b, We1, be1, We2, be2, Wn1, bn1, Wn2, bn2):
    ms = jnp.stack([m_u, m_v, m_y])
    hits = jnp.stack([edge_index_u[0], edge_index_v[0], edge_index_y[0]])
    sps = jnp.stack([edge_index_u[1], edge_index_v[1], edge_index_y[1]])
    hits = hits.astype(jnp.int32)
    sps = sps.astype(jnp.int32)
    pad = jnp.full((P, E_PAD - E), N, dtype=jnp.int32)
    hits_pad = jnp.concatenate([hits, pad], axis=1)
    sps_pad = jnp.concatenate([sps, pad], axis=1)

    w3a_pl = W3a.reshape(C, P, NF, SF).transpose(1, 0, 2, 3)
    we1_top = We1[:, :, :NF, :]
    we1_bot = We1[:, :, NF:, :]
    wk1 = jnp.concatenate([w3a_pl, we1_top], axis=-1)
    b3a_flat = b3a.reshape(1, C * SF)
    b3b_flat = b3b.reshape(1, C * SF)
    be1_flat = be1.reshape(P, C * SF)
    w2bd = (We2[:, :, :, 0][:, :, :, None]
            * jnp.eye(C, dtype=F32)[None, :, None, :]).reshape(P, C * SF, C)
    be2_flat = be2.reshape(P, C)
    ebd = jnp.repeat(jnp.eye(C, dtype=F32), SF, axis=1)

    t_all, a_all = _k1(ms, wk1)

    z = jnp.zeros((N, C * SF), dtype=F32)
    cnt = []
    for p in range(P):
        z = z + jax.ops.segment_sum(t_all[p][hits[p]], sps[p], num_segments=N)
        cnt.append(jax.ops.segment_sum(jnp.ones((E,), F32), hits[p],
                                       num_segments=N))
    cnt16 = jnp.broadcast_to(jnp.stack(cnt)[:, :, None], (P, N, 16))

    tab = _k3(z, b3a_flat, W3b, b3b_flat, we1_bot, be1_flat)

    hc = jnp.minimum(hits_pad, N - 1)
    sc = jnp.minimum(sps_pad, N - 1)
    ga = jnp.stack([jnp.take(a_all[p], hc[p], axis=0) for p in range(P)])
    gs = jnp.stack([jnp.take(tab[p], sc[p], axis=0) for p in range(P)])

    msg = _k5(ga, gs, w2bd, be2_flat, ebd)

    ssum = jnp.stack([
        jax.ops.segment_sum(msg[p][:E], hits[p], num_segments=N)
        for p in range(P)])

    return _k7(ms, ssum, cnt16, Wn1, bn1, Wn2, bn2)


# speedup vs baseline: 8.8557x; 8.8557x over previous
import functools

import jax
import jax.numpy as jnp
from jax import lax
from jax.experimental import pallas as pl
from jax.experimental.pallas import tpu as pltpu
from jax.experimental.pallas import tpu_sc as plsc

N = 50000
E = 100000
C = 5
NF = 64
SF = 16
P = 3

CK = 128
E_PAD = 102400
BN = 2000
BE = 2048
HALF = N // 2
TRASH = HALF

F32 = jnp.float32
INTERPRET = False


def _k1_body(m_ref, w_ref, t_ref, a_ref):
    x = m_ref[0]
    for c in range(C):
        y = jnp.dot(x[:, c, :], w_ref[0, c], preferred_element_type=F32)
        t_ref[0, :, c * SF:(c + 1) * SF] = y[:, :SF]
        a_ref[0, :, c * SF:(c + 1) * SF] = y[:, SF:]


def _k1(ms, wk1):
    return pl.pallas_call(
        _k1_body,
        grid=(P, N // BN),
        in_specs=[
            pl.BlockSpec((1, BN, C, NF), lambda p, i: (p, i, 0, 0)),
            pl.BlockSpec((1, C, NF, 2 * SF), lambda p, i: (p, 0, 0, 0)),
        ],
        out_specs=[
            pl.BlockSpec((1, BN, C * SF), lambda p, i: (p, i, 0)),
            pl.BlockSpec((1, BN, C * SF), lambda p, i: (p, i, 0)),
        ],
        out_shape=[
            jax.ShapeDtypeStruct((P, N, C * SF), F32),
            jax.ShapeDtypeStruct((P, N, C * SF), F32),
        ],
        interpret=INTERPRET,
    )(ms, wk1)


def _k3_body(z_ref, b3a_ref, w3b_ref, b3b_ref, wbot_ref, be1_ref, tab_ref):
    h = jnp.tanh(z_ref[...] + b3a_ref[...])
    msp = []
    for c in range(C):
        hc = h[:, c * SF:(c + 1) * SF]
        msp.append(jnp.tanh(
            jnp.dot(hc, w3b_ref[c], preferred_element_type=F32)
            + b3b_ref[0, c * SF:(c + 1) * SF]))
    msp = jnp.concatenate(msp, axis=1)
    for p in range(P):
        bp = []
        for c in range(C):
            bp.append(
                jnp.dot(msp[:, c * SF:(c + 1) * SF], wbot_ref[p, c],
                        preferred_element_type=F32)
                + be1_ref[p, c * SF:(c + 1) * SF])
        tab_ref[p, :, :C * SF] = jnp.concatenate(bp, axis=1)
        tab_ref[p, :, C * SF:] = msp


def _k3(z, b3a_flat, w3b, b3b_flat, wbot, be1_flat):
    return pl.pallas_call(
        _k3_body,
        grid=(N // BN,),
        in_specs=[
            pl.BlockSpec((BN, C * SF), lambda i: (i, 0)),
            pl.BlockSpec((1, C * SF), lambda i: (0, 0)),
            pl.BlockSpec((C, SF, SF), lambda i: (0, 0, 0)),
            pl.BlockSpec((1, C * SF), lambda i: (0, 0)),
            pl.BlockSpec((P, C, SF, SF), lambda i: (0, 0, 0, 0)),
            pl.BlockSpec((P, C * SF), lambda i: (0, 0)),
        ],
        out_specs=pl.BlockSpec((P, BN, 2 * C * SF), lambda i: (0, i, 0)),
        out_shape=jax.ShapeDtypeStruct((P, N, 2 * C * SF), F32),
        interpret=INTERPRET,
    )(z, b3a_flat, w3b, b3b_flat, wbot, be1_flat)


def _k5_body(ga_ref, gs_ref, w2_ref, be2_ref, ebd_ref, msg_ref):
    ga = ga_ref[0]
    gs = gs_ref[0]
    e1 = jnp.tanh(ga + gs[:, :C * SF])
    logits = jnp.dot(e1, w2_ref[0], preferred_element_type=F32) + be2_ref[0]
    mx = jnp.max(logits, axis=1, keepdims=True)
    ex = jnp.exp(logits - mx)
    att = ex / jnp.sum(ex, axis=1, keepdims=True)
    expand = jnp.dot(att, ebd_ref[...], preferred_element_type=F32)
    msg_ref[0] = expand * gs[:, C * SF:]


def _k5(ga, gs, w2bd, be2, ebd):
    return pl.pallas_call(
        _k5_body,
        grid=(P, E_PAD // BE),
        in_specs=[
            pl.BlockSpec((1, BE, C * SF), lambda p, i: (p, i, 0)),
            pl.BlockSpec((1, BE, 2 * C * SF), lambda p, i: (p, i, 0)),
            pl.BlockSpec((1, C * SF, C), lambda p, i: (p, 0, 0)),
            pl.BlockSpec((1, 1, C), lambda p, i: (p, 0, 0)),
            pl.BlockSpec((C, C * SF), lambda p, i: (0, 0)),
        ],
        out_specs=pl.BlockSpec((1, BE, C * SF), lambda p, i: (p, i, 0)),
        out_shape=jax.ShapeDtypeStruct((P, E_PAD, C * SF), F32),
        interpret=INTERPRET,
    )(ga, gs, w2bd, be2, ebd)


def _k7_body(m_ref, ss_ref, cnt_ref, wn1_ref, bn1_ref, wn2_ref, bn2_ref, o_ref):
    m = m_ref[0]
    ss = ss_ref[0]
    cnt = jnp.clip(cnt_ref[0][:, 0:1], 1.0, None)
    outs = []
    for c in range(C):
        mean_c = ss[:, c * SF:(c + 1) * SF] / cnt
        mcat = jnp.concatenate([m[:, c, :], mean_c], axis=1)
        h = jnp.tanh(jnp.dot(mcat, wn1_ref[0, c], preferred_element_type=F32)
                     + bn1_ref[0, c])
        h = jnp.tanh(jnp.dot(h, wn2_ref[0, c], preferred_element_type=F32)
                     + bn2_ref[0, c])
        outs.append(h[:, None, :])
    o_ref[0] = jnp.concatenate(outs, axis=1)


def _k7(ms, ssum, cnt, wn1, bn1, wn2, bn2):
    return pl.pallas_call(
        _k7_body,
        grid=(P, N // BN),
        in_specs=[
            pl.BlockSpec((1, BN, C, NF), lambda p, i: (p, i, 0, 0)),
            pl.BlockSpec((1, BN, C * SF), lambda p, i: (p, i, 0)),
            pl.BlockSpec((1, BN, 16), lambda p, i: (p, i, 0)),
            pl.BlockSpec((1, C, NF + SF, NF), lambda p, i: (p, 0, 0, 0)),
            pl.BlockSpec((1, C, NF), lambda p, i: (p, 0, 0)),
            pl.BlockSpec((1, C, NF, NF), lambda p, i: (p, 0, 0, 0)),
            pl.BlockSpec((1, C, NF), lambda p, i: (p, 0, 0)),
        ],
        out_specs=pl.BlockSpec((1, BN, C, NF), lambda p, i: (p, i, 0, 0)),
        out_shape=jax.ShapeDtypeStruct((P, N, C, NF), F32),
        interpret=INTERPRET,
    )(ms, ssum, cnt, wn1, bn1, wn2, bn2)


def kernel(m_u, m_v, m_y, edge_index_u, edge_index_v, edge_index_y,
           W3a, b3a, W3b, b3b, We1, be1, We2, be2, Wn1, bn1, Wn2, bn2):
    ms = jnp.stack([m_u, m_v, m_y])
    hits = jnp.stack([edge_index_u[0], edge_index_v[0], edge_index_y[0]])
    sps = jnp.stack([edge_index_u[1], edge_index_v[1], edge_index_y[1]])
    hits = hits.astype(jnp.int32)
    sps = sps.astype(jnp.int32)
    pad = jnp.full((P, E_PAD - E), N, dtype=jnp.int32)
    hits_pad = jnp.concatenate([hits, pad], axis=1)
    sps_pad = jnp.concatenate([sps, pad], axis=1)

    w3a_pl = W3a.reshape(C, P, NF, SF).transpose(1, 0, 2, 3)
    we1_top = We1[:, :, :NF, :]
    we1_bot = We1[:, :, NF:, :]
    wk1 = jnp.concatenate([w3a_pl, we1_top], axis=-1)
    b3a_flat = b3a.reshape(1, C * SF)
    b3b_flat = b3b.reshape(1, C * SF)
    be1_flat = be1.reshape(P, C * SF)
    w2bd = (We2[:, :, :, 0][:, :, :, None]
            * jnp.eye(C, dtype=F32)[None, :, None, :]).reshape(P, C * SF, C)
    be2_flat = be2.reshape(P, 1, C)
    ebd = jnp.repeat(jnp.eye(C, dtype=F32), SF, axis=1)

    t_all, a_all = _k1(ms, wk1)

    z = jnp.zeros((N, C * SF), dtype=F32)
    cnt = []
    for p in range(P):
        z = z + jax.ops.segment_sum(t_all[p][hits[p]], sps[p], num_segments=N)
        cnt.append(jax.ops.segment_sum(jnp.ones((E,), F32), hits[p],
                                       num_segments=N))
    cnt16 = jnp.broadcast_to(jnp.stack(cnt)[:, :, None], (P, N, 16))

    tab = _k3(z, b3a_flat, W3b, b3b_flat, we1_bot, be1_flat)

    hc = jnp.minimum(hits_pad, N - 1)
    sc = jnp.minimum(sps_pad, N - 1)
    ga = jnp.stack([jnp.take(a_all[p], hc[p], axis=0) for p in range(P)])
    gs = jnp.stack([jnp.take(tab[p], sc[p], axis=0) for p in range(P)])

    msg = _k5(ga, gs, w2bd, be2_flat, ebd)

    ssum = jnp.stack([
        jax.ops.segment_sum(msg[p][:E], hits[p], num_segments=N)
        for p in range(P)])

    return _k7(ms, ssum, cnt16, Wn1, bn1, Wn2, bn2)
